# baseline (speedup 1.0000x reference)
import functools

import jax
import jax.numpy as jnp
from jax.experimental import pallas as pl

LATENT = 256
KCB = 256
NCB = 3


def _patches(xh):
    B, H, W, C = xh.shape
    P, Q = H // 2, W // 2
    xp = jnp.pad(xh, ((0, 0), (1, 2), (1, 2), (0, 0)))
    taps = []
    for kh in range(4):
        for kw in range(4):
            taps.append(xp[:, kh:kh + 2 * P - 1:2, kw:kw + 2 * Q - 1:2, :])
    return jnp.concatenate(taps, axis=-1)


def _conv_mm_kernel(p_ref, w_ref, b_ref, o_ref, *, P, Q, KK, O, relu):
    xg = p_ref[0].reshape(P * Q, KK)
    w = w_ref[...]

    def dd(a, b):
        return jax.lax.dot_general(a, b, (((1,), (0,)), ((), ())),
                                   preferred_element_type=jnp.float32)

    nchunk = max(1, KK // 256)
    if nchunk == 1:
        acc = dd(xg, w)
    else:
        acc = dd(xg[:, 0:256], w[0:256])
        for i in range(1, nchunk):
            acc = acc + dd(xg[:, 256 * i:256 * (i + 1)], w[256 * i:256 * (i + 1)])
    acc = acc + b_ref[0]
    if relu:
        acc = jnp.maximum(acc, 0.0)
    o_ref[0] = acc.reshape(P, Q, O)


def _conv(xh, W, b, relu):
    B, H, Wd, C = xh.shape
    P, Q = H // 2, Wd // 2
    O = W.shape[0]
    KK = 16 * C
    p = _patches(xh)
    Wf = jnp.transpose(W, (2, 3, 1, 0)).reshape(KK, O)
    return pl.pallas_call(
        functools.partial(_conv_mm_kernel, P=P, Q=Q, KK=KK, O=O, relu=relu),
        grid=(B,),
        in_specs=[
            pl.BlockSpec((1, P, Q, KK), lambda b: (b, 0, 0, 0)),
            pl.BlockSpec((KK, O), lambda b: (0, 0)),
            pl.BlockSpec((1, O), lambda b: (0, 0)),
        ],
        out_specs=pl.BlockSpec((1, P, Q, O), lambda b: (b, 0, 0, 0)),
        out_shape=jax.ShapeDtypeStruct((B, P, Q, O), jnp.float32),
    )(p, Wf, b.reshape(1, O))


def _rvq_kernel(lat_ref, cb_ref, idx_ref):
    r = lat_ref[...]
    M, D = r.shape
    iota = jax.lax.broadcasted_iota(jnp.int32, (M, KCB), 1)
    for i in range(NCB):
        cb = cb_ref[i]
        cb2 = jnp.sum(cb * cb, axis=1)
        r2 = jnp.sum(r * r, axis=1, keepdims=True)
        prod = jax.lax.dot_general(r, cb, (((1,), (1,)), ((), ())),
                                   preferred_element_type=jnp.float32)
        d = r2 - 2.0 * prod + cb2[None, :]
        m = jnp.min(d, axis=1, keepdims=True)
        idx = jnp.min(jnp.where(d == m, iota, KCB), axis=1)
        idx_ref[i, :] = idx
        if i + 1 < NCB:
            oh = (iota == idx[:, None]).astype(jnp.float32)
            q = jax.lax.dot_general(oh, cb, (((1,), (0,)), ((), ())),
                                    preferred_element_type=jnp.float32,
                                    precision=jax.lax.Precision.HIGHEST)
            r = r - q


def kernel(x, W1, b1, W2, b2, W3, b3, W4, b4, codebooks):
    B = x.shape[0]
    xh = jnp.transpose(x, (0, 2, 3, 1))
    h1 = _conv(xh, W1, b1, True)
    h2 = _conv(h1, W2, b2, True)
    h3 = _conv(h2, W3, b3, True)
    h4 = _conv(h3, W4, b4, False)

    lat = h4.reshape(B * 196, LATENT)
    idx = pl.pallas_call(
        _rvq_kernel,
        in_specs=[
            pl.BlockSpec((B * 196, LATENT), lambda: (0, 0)),
            pl.BlockSpec((NCB, KCB, LATENT), lambda: (0, 0, 0)),
        ],
        out_specs=pl.BlockSpec((NCB, B * 196), lambda: (0, 0)),
        out_shape=jax.ShapeDtypeStruct((NCB, B * 196), jnp.int32),
    )(lat, codebooks)
    return jnp.transpose(idx, (1, 0)).reshape(B, 196, NCB)


# speedup vs baseline: 1.8229x; 1.8229x over previous
import functools

import jax
import jax.numpy as jnp
from jax.experimental import pallas as pl

LATENT = 256
KCB = 256
NCB = 3

_PH = ((1, 0), (0, 0), (1, 1), (0, 1))


def _zphase(y):
    B, H, W, C = y.shape
    P, Q = H // 2, W // 2
    yr = y.reshape(B, P, 2, Q, 2, C)
    blocks = []
    for ah in (0, 1):
        for aw in (1, 0):
            p = yr[:, :, ah, :, aw, :]
            ph = (1, 0) if ah else (0, 1)
            pw = (1, 0) if aw else (0, 1)
            blocks.append(jnp.pad(p, ((0, 0), ph, pw, (0, 0))))
    return jnp.concatenate(blocks, axis=-1)


def _conv_phase_kernel(z_ref, w_ref, b_ref, o_ref, *, P, Q, C, O, relu):
    w = w_ref[...]

    def dd(a, b):
        return jax.lax.dot_general(a, b, (((1,), (0,)), ((), ())),
                                   preferred_element_type=jnp.float32)

    acc = None
    for kh in range(4):
        ah, oh = _PH[kh]
        lo = ah * 2 * C
        if C < 128:
            s0 = z_ref[0, oh:oh + P, 0:Q, lo:lo + 2 * C]
            s1 = z_ref[0, oh:oh + P, 1:Q + 1, lo:lo + 2 * C]
            xg = jnp.concatenate([s0, s1], axis=-1).reshape(P * Q, 4 * C)
            term = dd(xg, w[4 * C * kh:4 * C * (kh + 1)])
            acc = term if acc is None else acc + term
        else:
            nh = (2 * C) // 256
            for ow in (0, 1):
                xg = z_ref[0, oh:oh + P, ow:ow + Q, lo:lo + 2 * C]
                xg = xg.reshape(P * Q, 2 * C)
                for h in range(nh):
                    kbase = 4 * C * kh + 2 * C * ow + 256 * h
                    term = dd(xg[:, 256 * h:256 * (h + 1)],
                              w[kbase:kbase + 256])
                    acc = term if acc is None else acc + term
    acc = acc + b_ref[0]
    if relu:
        acc = jnp.maximum(acc, 0.0)
    o_ref[0] = acc.reshape(P, Q, O)


def _conv_phase(y, W, b, relu):
    B, H, Wd, C = y.shape
    P, Q = H // 2, Wd // 2
    O = W.shape[0]
    z = _zphase(y)
    Wf = jnp.transpose(W, (2, 3, 1, 0)).reshape(16 * C, O)
    return pl.pallas_call(
        functools.partial(_conv_phase_kernel, P=P, Q=Q, C=C, O=O, relu=relu),
        grid=(B,),
        in_specs=[
            pl.BlockSpec((1, P + 1, Q + 1, 4 * C), lambda b: (b, 0, 0, 0)),
            pl.BlockSpec((16 * C, O), lambda b: (0, 0)),
            pl.BlockSpec((1, O), lambda b: (0, 0)),
        ],
        out_specs=pl.BlockSpec((1, P, Q, O), lambda b: (b, 0, 0, 0)),
        out_shape=jax.ShapeDtypeStruct((B, P, Q, O), jnp.float32),
    )(z, Wf, b.reshape(1, O))


def _conv1_patches(x):
    xh = jnp.transpose(x, (0, 2, 3, 1))
    xp = jnp.pad(xh, ((0, 0), (1, 2), (1, 2), (0, 0)))
    taps = []
    for kh in range(4):
        for kw in range(4):
            taps.append(xp[:, kh:kh + 223:2, kw:kw + 223:2, :])
    return jnp.concatenate(taps, axis=-1)


def _conv1_kernel(p_ref, w_ref, b_ref, o_ref):
    xg = p_ref[0].reshape(112 * 112, 48)
    acc = jax.lax.dot_general(xg, w_ref[...], (((1,), (0,)), ((), ())),
                              preferred_element_type=jnp.float32)
    acc = jnp.maximum(acc + b_ref[0], 0.0)
    o_ref[0] = acc.reshape(112, 112, 64)


def _rvq_kernel(lat_ref, cb_ref, idx_ref):
    r = lat_ref[...]
    M, D = r.shape
    iota = jax.lax.broadcasted_iota(jnp.int32, (M, KCB), 1)
    for i in range(NCB):
        cb = cb_ref[i]
        cb2 = jnp.sum(cb * cb, axis=1)
        r2 = jnp.sum(r * r, axis=1, keepdims=True)
        prod = jax.lax.dot_general(r, cb, (((1,), (1,)), ((), ())),
                                   preferred_element_type=jnp.float32)
        d = r2 - 2.0 * prod + cb2[None, :]
        m = jnp.min(d, axis=1, keepdims=True)
        idx = jnp.min(jnp.where(d == m, iota, KCB), axis=1)
        idx_ref[i, :] = idx
        if i + 1 < NCB:
            oh = (iota == idx[:, None]).astype(jnp.float32)
            q = jax.lax.dot_general(oh, cb, (((1,), (0,)), ((), ())),
                                    preferred_element_type=jnp.float32,
                                    precision=jax.lax.Precision.HIGHEST)
            r = r - q


def kernel(x, W1, b1, W2, b2, W3, b3, W4, b4, codebooks):
    B = x.shape[0]

    p1 = _conv1_patches(x)
    W1f = jnp.transpose(W1, (2, 3, 1, 0)).reshape(48, 64)
    h1 = pl.pallas_call(
        _conv1_kernel,
        grid=(B,),
        in_specs=[
            pl.BlockSpec((1, 112, 112, 48), lambda b: (b, 0, 0, 0)),
            pl.BlockSpec((48, 64), lambda b: (0, 0)),
            pl.BlockSpec((1, 64), lambda b: (0, 0)),
        ],
        out_specs=pl.BlockSpec((1, 112, 112, 64), lambda b: (b, 0, 0, 0)),
        out_shape=jax.ShapeDtypeStruct((B, 112, 112, 64), jnp.float32),
    )(p1, W1f, b1.reshape(1, 64))

    h2 = _conv_phase(h1, W2, b2, True)
    h3 = _conv_phase(h2, W3, b3, True)
    h4 = _conv_phase(h3, W4, b4, False)

    lat = h4.reshape(B * 196, LATENT)
    idx = pl.pallas_call(
        _rvq_kernel,
        in_specs=[
            pl.BlockSpec((B * 196, LATENT), lambda: (0, 0)),
            pl.BlockSpec((NCB, KCB, LATENT), lambda: (0, 0, 0)),
        ],
        out_specs=pl.BlockSpec((NCB, B * 196), lambda: (0, 0)),
        out_shape=jax.ShapeDtypeStruct((NCB, B * 196), jnp.int32),
    )(lat, codebooks)
    return jnp.transpose(idx, (1, 0)).reshape(B, 196, NCB)
